# baseline (speedup 1.0000x reference)
import functools

import jax
import jax.numpy as jnp
from jax import lax
from jax.experimental import pallas as pl
from jax.experimental.pallas import tpu as pltpu
from jax.experimental.pallas import tpu_sc as plsc

B = 4096
NEG = 4
D = 64
V = 100000
VP = V // 2
NC = 2
NS = 16
NW = NC * NS
BPW = B // NW
L = 16
GROUPS = BPW // L
NOUT = 1 + NEG
NBLK = V // 128
NTAIL = V - NBLK * 128

_mesh = plsc.VectorSubcoreMesh(core_axis_name="c", subcore_axis_name="s")


@functools.partial(
    pl.kernel,
    mesh=_mesh,
    compiler_params=pltpu.CompilerParams(needs_layout_passes=False,
                                         use_tc_tiling_on_sc=True,
                                         internal_scratch_in_bytes=8192),
    out_type=(jax.ShapeDtypeStruct((VP, 2 * D), jnp.float32),
              jax.ShapeDtypeStruct((VP, 2 * D), jnp.float32)),
    scratch_types=[
        pltpu.VMEM((D, 512), jnp.float32),
        pltpu.VMEM((D, 512), jnp.float32),
        pltpu.VMEM((256, 2 * D + 1), jnp.float32),
        pltpu.SemaphoreType.DMA,
    ],
)
def _format_sc(ut_t, it_t, ut_tail, it_tail, utd, itd,
               slab0, slab1, obuf, sem_in):
    wid = lax.axis_index("s") * NC + lax.axis_index("c")
    iota = lax.iota(jnp.int32, L)
    slabs = (slab0, slab1)
    srcs, dsts = (ut_t, it_t), (utd, itd)
    W = 512
    NBW = (NBLK * 128) // W
    REM0 = NBW * W
    NSLOT = 2 * ((NBW + NW - 1) // NW)

    def cb_of(s):
        return wid + (s // 2) * NW

    def in_copy(s, par, buf):
        return pltpu.make_async_copy(
            srcs[par].at[:, pl.ds(cb_of(s) * W, W)], slabs[buf], sem_in)

    rvecs = [(16 * g + iota) >> 1 for g in range(W // L)]
    cbase = (iota & 1) * D

    def transpose_slab(slab, ngroups):
        def d_body(d, _):
            cvec = cbase + d
            for g0 in range(0, ngroups, 8):
                vals = [slab[d, pl.ds(16 * (g0 + g), L)] for g in range(8)]
                for g in range(8):
                    plsc.store_scatter(obuf, [rvecs[g0 + g], cvec], vals[g])
            return 0
        lax.fori_loop(0, D, d_body, 0)

    @pl.when(cb_of(0) < NBW)
    def _():
        in_copy(0, 0, 0).start()

    def pair_body(i, _):
        for b in (0, 1):
            s = 2 * i + b
            valid = cb_of(s) < NBW

            @pl.when(valid)
            def _():
                in_copy(s, b, b).wait()
                @pl.when(cb_of(s + 1) < NBW)
                def _():
                    in_copy(s + 1, 1 - b, 1 - b).start()
                transpose_slab(slabs[b], W // L)
                pltpu.sync_copy(
                    obuf.at[:, pl.ds(0, 2 * D)],
                    dsts[b].at[pl.ds(cb_of(s) * (W // 2), W // 2)])
        return 0

    lax.fori_loop(0, NSLOT // 2, pair_body, 0)

    def do_small(get_src, dst, row0):
        pltpu.sync_copy(get_src, slab0.at[:, pl.ds(0, 128)])
        def d_body(d, _):
            cvec = cbase + d
            vals = [slab0[d, pl.ds(16 * g, L)] for g in range(8)]
            for g in range(8):
                plsc.store_scatter(obuf, [rvecs[g], cvec], vals[g])
            return 0
        lax.fori_loop(0, D, d_body, 0)
        pltpu.sync_copy(obuf.at[pl.ds(0, 64), pl.ds(0, 2 * D)],
                        dst.at[pl.ds(row0, 64)])

    @pl.when(wid == NW - 4)
    def _():
        do_small(ut_t.at[:, pl.ds(REM0, 128)], utd, REM0 // 2)

    @pl.when(wid == NW - 3)
    def _():
        do_small(it_t.at[:, pl.ds(REM0, 128)], itd, REM0 // 2)

    @pl.when(wid == NW - 2)
    def _():
        do_small(ut_tail, utd, (V - 128) // 2)

    @pl.when(wid == NW - 1)
    def _():
        do_small(it_tail, itd, (V - 128) // 2)


@functools.partial(
    pl.kernel,
    mesh=_mesh,
    compiler_params=pltpu.CompilerParams(needs_layout_passes=False,
                                         use_tc_tiling_on_sc=False),
    out_type=jax.ShapeDtypeStruct((B, NOUT), jnp.float32),
    scratch_types=[
        pltpu.VMEM((BPW,), jnp.int32),
        pltpu.VMEM((BPW,), jnp.int32),
        pltpu.VMEM((NEG, BPW), jnp.int32),
        pltpu.VMEM((BPW,), jnp.int32),
        pltpu.VMEM((BPW,), jnp.int32),
        pltpu.VMEM((NEG, BPW), jnp.int32),
        pltpu.VMEM((BPW, 2 * D), jnp.float32),
        pltpu.VMEM((BPW, 2 * D), jnp.float32),
        pltpu.VMEM((NEG * BPW, 2 * D), jnp.float32),
        pltpu.VMEM((BPW, NOUT), jnp.float32),
        pltpu.SemaphoreType.DMA,
    ],
)
def _bpr_sc(user_hbm, pos_hbm, n0_hbm, n1_hbm, n2_hbm, n3_hbm,
            utd_hbm, itd_hbm, out_hbm,
            uidx, pidx, nidx, ubx, pbx, nbx, urows, prows, nrows, oblk, sem):
    wid = lax.axis_index("s") * NC + lax.axis_index("c")
    base = wid * BPW
    iota = lax.iota(jnp.int32, L)

    idx_copies = [
        pltpu.async_copy(user_hbm.at[pl.ds(base, BPW)], uidx, sem),
        pltpu.async_copy(pos_hbm.at[pl.ds(base, BPW)], pidx, sem),
    ] + [
        pltpu.async_copy(n_hbm.at[pl.ds(base, BPW)], nidx.at[j], sem)
        for j, n_hbm in enumerate((n0_hbm, n1_hbm, n2_hbm, n3_hbm))
    ]
    for c in idx_copies:
        c.wait()

    def bx_body(g, _):
        s = pl.ds(g * L, L)
        ubx[s] = lax.shift_right_logical(uidx[s], 1)
        pbx[s] = lax.shift_right_logical(pidx[s], 1)
        for j in range(NEG):
            nbx[j, s] = lax.shift_right_logical(nidx[j, s], 1)
        return 0
    lax.fori_loop(0, GROUPS, bx_body, 0)

    copies = [
        pltpu.async_copy(utd_hbm.at[ubx], urows, sem),
        pltpu.async_copy(itd_hbm.at[pbx], prows, sem),
    ]
    for j in range(NEG):
        copies.append(
            pltpu.async_copy(itd_hbm.at[nbx.at[j]],
                             nrows.at[pl.ds(j * BPW, BPW)], sem))
    for c in copies:
        c.wait()

    last_lane = iota == (L - 1)

    def group_body(g, _):
        s = pl.ds(g * L, L)
        uc = (uidx[s] & 1) * D
        pc = (pidx[s] & 1) * D
        ncs = [(nidx[j, s] & 1) * D for j in range(NEG)]

        for l in range(L):
            b = g * L + l
            uo, po = uc[l], pc[l]
            uvs = [urows[b, pl.ds(uo + L * k, L)] for k in range(4)]
            pvs = [prows[b, pl.ds(po + L * k, L)] for k in range(4)]
            nvss = [[nrows[j * BPW + b, pl.ds(ncs[j][l] + L * k, L)]
                     for k in range(4)] for j in range(NEG)]
            prods = [pvs] + nvss
            sums = [plsc.cumsum(
                (uvs[0] * w[0] + uvs[1] * w[1]) +
                (uvs[2] * w[2] + uvs[3] * w[3])) for w in prods]
            bv = jnp.full((L,), b, jnp.int32)
            for c in range(NOUT):
                plsc.store_scatter(oblk, [bv, jnp.full((L,), c, jnp.int32)],
                                   sums[c], mask=last_lane)
        return 0

    lax.fori_loop(0, GROUPS, group_body, 0)
    pltpu.sync_copy(oblk, out_hbm.at[pl.ds(base, BPW)])


def kernel(user, pos_item, neg_item, user_table, item_table):
    utt, itt = user_table.T, item_table.T
    utd, itd = _format_sc(utt, itt, utt[:, V - 128:], itt[:, V - 128:])
    negs = [neg_item[:, j] for j in range(NEG)]
    return _bpr_sc(user, pos_item, *negs, utd, itd)


# speedup vs baseline: 2.3442x; 2.3442x over previous
import functools

import jax
import jax.numpy as jnp
from jax import lax
from jax.experimental import pallas as pl
from jax.experimental.pallas import tpu as pltpu
from jax.experimental.pallas import tpu_sc as plsc

B = 4096
NEG = 4
D = 64
NC = 2
NS = 16
NW = NC * NS
BPW = B // NW
L = 16
GROUPS = BPW // L
NOUT = 1 + NEG

_mesh = plsc.VectorSubcoreMesh(core_axis_name="c", subcore_axis_name="s")


@functools.partial(
    pl.kernel,
    mesh=_mesh,
    compiler_params=pltpu.CompilerParams(needs_layout_passes=False,
                                         use_tc_tiling_on_sc=False),
    out_type=jax.ShapeDtypeStruct((B, NOUT), jnp.float32),
    scratch_types=[
        pltpu.VMEM((BPW,), jnp.int32),
        pltpu.VMEM((BPW,), jnp.int32),
        pltpu.VMEM((NEG, BPW), jnp.int32),
        pltpu.VMEM((BPW, D), jnp.float32),
        pltpu.VMEM((BPW, D), jnp.float32),
        pltpu.VMEM((NEG * BPW, D), jnp.float32),
        pltpu.VMEM((BPW, NOUT), jnp.float32),
        pltpu.SemaphoreType.DMA,
    ],
)
def _bpr_sc(user_hbm, pos_hbm, n0_hbm, n1_hbm, n2_hbm, n3_hbm,
            utab_hbm, itab_hbm, out_hbm,
            uidx, pidx, nidx, urows, prows, nrows, oblk, sem):
    wid = lax.axis_index("s") * NC + lax.axis_index("c")
    base = wid * BPW
    iota = lax.iota(jnp.int32, L)

    idx_copies = [
        pltpu.async_copy(user_hbm.at[pl.ds(base, BPW)], uidx, sem),
        pltpu.async_copy(pos_hbm.at[pl.ds(base, BPW)], pidx, sem),
    ] + [
        pltpu.async_copy(n_hbm.at[pl.ds(base, BPW)], nidx.at[j], sem)
        for j, n_hbm in enumerate((n0_hbm, n1_hbm, n2_hbm, n3_hbm))
    ]
    for c in idx_copies:
        c.wait()

    copies = [
        pltpu.async_copy(utab_hbm.at[uidx], urows, sem),
        pltpu.async_copy(itab_hbm.at[pidx], prows, sem),
    ] + [
        pltpu.async_copy(itab_hbm.at[nidx.at[j]],
                         nrows.at[pl.ds(j * BPW, BPW)], sem)
        for j in range(NEG)
    ]
    for c in copies:
        c.wait()

    last_lane = iota == (L - 1)

    def group_body(g, _):
        for l in range(L):
            b = g * L + l
            uvs = [urows[b, pl.ds(L * k, L)] for k in range(4)]
            pvs = [prows[b, pl.ds(L * k, L)] for k in range(4)]
            nvss = [[nrows[j * BPW + b, pl.ds(L * k, L)] for k in range(4)]
                    for j in range(NEG)]
            sums = [plsc.cumsum(
                (uvs[0] * w[0] + uvs[1] * w[1]) +
                (uvs[2] * w[2] + uvs[3] * w[3]))
                for w in [pvs] + nvss]
            bv = jnp.full((L,), b, jnp.int32)
            for c in range(NOUT):
                plsc.store_scatter(oblk, [bv, jnp.full((L,), c, jnp.int32)],
                                   sums[c], mask=last_lane)
        return 0

    lax.fori_loop(0, GROUPS, group_body, 0)
    pltpu.sync_copy(oblk, out_hbm.at[pl.ds(base, BPW)])


def kernel(user, pos_item, neg_item, user_table, item_table):
    negs = [neg_item[:, j] for j in range(NEG)]
    return _bpr_sc(user, pos_item, *negs, user_table, item_table)
